# baseline (speedup 1.0000x reference)
import jax
import jax.numpy as jnp
from jax.experimental import pallas as pl
from jax.experimental.pallas import tpu as pltpu

NHEADS = 12
NPTS = 8
HDIM = 64
CHUNK = 256


def _fused_kernel(qt_ref, qtb_ref, rpt_ref, wcat_ref, bcat_ref, wvb_ref, bvb_ref,
                  wout_ref, bout_ref, out_ref, scr_ref):
    n = pl.program_id(0)
    h = pl.program_id(1)
    L = qt_ref.shape[2]
    gw = 32

    @pl.when(n < pl.num_programs(0) - 1)
    def _tents():
        qt = qt_ref[0]
        rt = jnp.dot(wcat_ref[0], qt, preferred_element_type=jnp.float32) + bcat_ref[0]
        valt = (jnp.dot(wvb_ref[0], qtb_ref[0], preferred_element_type=jnp.float32)
                + bvb_ref[0])
        xft = rt[0:NPTS, :] + rpt_ref[0, 0:1, :]
        yft = rt[NPTS:2 * NPTS, :] + rpt_ref[0, 1:2, :]
        logits = rt[2 * NPTS:3 * NPTS, :]
        m = jnp.max(logits, axis=0, keepdims=True)
        e = jnp.exp(logits - m)
        attnt = e / jnp.sum(e, axis=0, keepdims=True)

        g = jax.lax.broadcasted_iota(jnp.int32, (gw, 1), 0).astype(jnp.float32)
        txa_list = []
        ty_list = []
        for p in range(NPTS):
            ap = attnt[p:p + 1, :]
            txa = jnp.maximum(ap - ap * jnp.abs(g - xft[p:p + 1, :]), 0.0)
            ty = jnp.maximum(1.0 - jnp.abs(g - yft[p:p + 1, :]), 0.0)
            txa_list.append(txa.astype(jnp.bfloat16))
            ty_list.append(ty.astype(jnp.bfloat16))
        valb = valt.astype(jnp.bfloat16)
        slabs = []
        for j in range(0, L // gw, 2):
            s0 = None
            s1 = None
            for p in range(NPTS):
                x = txa_list[p]
                t0 = x * ty_list[p][j:j + 1, :]
                t1 = x * ty_list[p][j + 1:j + 2, :]
                s0 = t0 if s0 is None else s0 + t0
                s1 = t1 if s1 is None else s1 + t1
            slabs.append(s0)
            slabs.append(s1)
        acc = jnp.concatenate(slabs, axis=0)
        sampledt = jnp.dot(valb, acc, preferred_element_type=jnp.float32)
        scr_ref[jax.lax.rem(n, 2), pl.ds(h * HDIM, HDIM), :] = (
            sampledt.astype(jnp.bfloat16))

    @pl.when((n > 0) & (h == 0))
    def _project():
        prev = scr_ref[jax.lax.rem(n + 1, 2)]
        outv = jnp.dot(wout_ref[...], prev, preferred_element_type=jnp.float32)
        out_ref[0] = outv + bout_ref[...]


def kernel(query, reference_points, W_off, b_off, W_attn, b_attn, W_val, b_val, W_out, b_out):
    N, H, W, E = query.shape
    L = H * W
    qt = query.reshape(N, L, E).transpose(0, 2, 1)
    Wv = W_val.reshape(E, NHEADS, HDIM).transpose(1, 0, 2)
    Wo2 = W_off.reshape(E, NHEADS, NPTS, 2)
    Wox = float(W) * Wo2[..., 0].transpose(1, 0, 2)
    Woy = float(H) * Wo2[..., 1].transpose(1, 0, 2)
    Wa = W_attn.reshape(E, NHEADS, NPTS).transpose(1, 0, 2)
    Wcat = jnp.concatenate([Wox, Woy, Wa], axis=2)
    WcatT = Wcat.transpose(0, 2, 1)
    WvT = Wv.transpose(0, 2, 1).astype(jnp.bfloat16)
    bvb = b_val.reshape(NHEADS, HDIM)[:, :, None]
    qtb = query.reshape(N, L, E).transpose(0, 2, 1).astype(jnp.bfloat16)
    bo2 = b_off.reshape(NHEADS, NPTS, 2)
    bcat = jnp.concatenate([float(W) * bo2[..., 0], float(H) * bo2[..., 1],
                            b_attn.reshape(NHEADS, NPTS)], axis=1)[:, :, None]
    rpt = (reference_points.reshape(N, L, 2) * jnp.array([W, H], jnp.float32)
           - 0.5).transpose(0, 2, 1)
    WoutF = W_out.T.astype(jnp.bfloat16)
    boutT = b_out.reshape(E, 1)

    outT = pl.pallas_call(
        _fused_kernel,
        grid=(N + 1, NHEADS),
        in_specs=[
            pl.BlockSpec((1, E, L), lambda n, h: (jnp.minimum(n, N - 1), 0, 0)),
            pl.BlockSpec((1, E, L), lambda n, h: (jnp.minimum(n, N - 1), 0, 0)),
            pl.BlockSpec((1, 2, L), lambda n, h: (jnp.minimum(n, N - 1), 0, 0)),
            pl.BlockSpec((1, 3 * NPTS, E), lambda n, h: (h, 0, 0)),
            pl.BlockSpec((1, 3 * NPTS, 1), lambda n, h: (h, 0, 0)),
            pl.BlockSpec((1, HDIM, E), lambda n, h: (h, 0, 0)),
            pl.BlockSpec((1, HDIM, 1), lambda n, h: (h, 0, 0)),
            pl.BlockSpec((E, E), lambda n, h: (0, 0)),
            pl.BlockSpec((E, 1), lambda n, h: (0, 0)),
        ],
        out_specs=pl.BlockSpec((1, E, L), lambda n, h: (jnp.maximum(n - 1, 0), 0, 0)),
        out_shape=jax.ShapeDtypeStruct((N, E, L), jnp.float32),
        scratch_shapes=[pltpu.VMEM((2, E, L), jnp.bfloat16)],
        compiler_params=pltpu.CompilerParams(
            dimension_semantics=("arbitrary", "arbitrary")),
    )(qt, qtb, rpt, WcatT, bcat, WvT, bvb, WoutF, boutT)
    return outT.transpose(0, 2, 1).reshape(N, H, W, E)


# speedup vs baseline: 1.0622x; 1.0622x over previous
import jax
import jax.numpy as jnp
from jax.experimental import pallas as pl
from jax.experimental.pallas import tpu as pltpu

NHEADS = 12
NPTS = 8
HDIM = 64
CHUNK = 256


def _fused_kernel(qt_ref, rpt_ref, wcat_ref, bcat_ref, wout_ref, bout_ref,
                  out_ref, scr_ref):
    n = pl.program_id(0)
    h = pl.program_id(1)
    L = qt_ref.shape[2]
    gw = 32

    @pl.when(n < pl.num_programs(0) - 1)
    def _tents():
        qt = qt_ref[0]
        rt = jnp.dot(wcat_ref[0], qt, preferred_element_type=jnp.float32) + bcat_ref[0]
        valt = rt[0:HDIM, :]
        xft = rt[HDIM:HDIM + NPTS, :] + rpt_ref[0, 0:1, :]
        yft = rt[HDIM + NPTS:HDIM + 2 * NPTS, :] + rpt_ref[0, 1:2, :]
        logits = rt[HDIM + 2 * NPTS:HDIM + 3 * NPTS, :]
        m = jnp.max(logits, axis=0, keepdims=True)
        e = jnp.exp(logits - m)
        attnt = e / jnp.sum(e, axis=0, keepdims=True)

        g = jax.lax.broadcasted_iota(jnp.int32, (gw, 1), 0).astype(jnp.float32)
        txa_list = []
        ty_list = []
        for p in range(NPTS):
            ap = attnt[p:p + 1, :]
            txa = jnp.maximum(ap - ap * jnp.abs(g - xft[p:p + 1, :]), 0.0)
            ty = jnp.maximum(1.0 - jnp.abs(g - yft[p:p + 1, :]), 0.0)
            txa_list.append(txa.astype(jnp.bfloat16))
            ty_list.append(ty.astype(jnp.bfloat16))
        valb = valt.astype(jnp.bfloat16)
        slabs = []
        for j in range(0, L // gw, 2):
            s0 = None
            s1 = None
            for p in range(NPTS):
                x = txa_list[p]
                t0 = x * ty_list[p][j:j + 1, :]
                t1 = x * ty_list[p][j + 1:j + 2, :]
                s0 = t0 if s0 is None else s0 + t0
                s1 = t1 if s1 is None else s1 + t1
            slabs.append(s0)
            slabs.append(s1)
        acc = jnp.concatenate(slabs, axis=0)
        sampledt = jnp.dot(valb, acc, preferred_element_type=jnp.float32)
        scr_ref[jax.lax.rem(n, 2), pl.ds(h * HDIM, HDIM), :] = (
            sampledt.astype(jnp.bfloat16))

    @pl.when((n > 0) & (h < 2))
    def _project():
        E = out_ref.shape[1]
        prev = scr_ref[jax.lax.rem(n + 1, 2)]
        rows = pl.ds(h * (E // 2), E // 2)
        outv = jnp.dot(wout_ref[rows, :], prev,
                       preferred_element_type=jnp.float32)
        out_ref[0, rows, :] = outv + bout_ref[rows, :]


def kernel(query, reference_points, W_off, b_off, W_attn, b_attn, W_val, b_val, W_out, b_out):
    N, H, W, E = query.shape
    L = H * W
    qt = query.reshape(N, L, E).transpose(0, 2, 1)
    Wv = W_val.reshape(E, NHEADS, HDIM).transpose(1, 0, 2)
    Wo2 = W_off.reshape(E, NHEADS, NPTS, 2)
    Wox = float(W) * Wo2[..., 0].transpose(1, 0, 2)
    Woy = float(H) * Wo2[..., 1].transpose(1, 0, 2)
    Wa = W_attn.reshape(E, NHEADS, NPTS).transpose(1, 0, 2)
    Wcat = jnp.concatenate([Wv, Wox, Woy, Wa], axis=2)
    WcatT = Wcat.transpose(0, 2, 1)
    bo2 = b_off.reshape(NHEADS, NPTS, 2)
    bcat = jnp.concatenate([b_val.reshape(NHEADS, HDIM),
                            float(W) * bo2[..., 0], float(H) * bo2[..., 1],
                            b_attn.reshape(NHEADS, NPTS)], axis=1)[:, :, None]
    rpt = (reference_points.reshape(N, L, 2) * jnp.array([W, H], jnp.float32)
           - 0.5).transpose(0, 2, 1)
    WoutF = W_out.T.astype(jnp.bfloat16)
    boutT = b_out.reshape(E, 1)

    outT = pl.pallas_call(
        _fused_kernel,
        grid=(N + 1, NHEADS),
        in_specs=[
            pl.BlockSpec((1, E, L), lambda n, h: (jnp.minimum(n, N - 1), 0, 0)),
            pl.BlockSpec((1, 2, L), lambda n, h: (jnp.minimum(n, N - 1), 0, 0)),
            pl.BlockSpec((1, HDIM + 3 * NPTS, E), lambda n, h: (h, 0, 0)),
            pl.BlockSpec((1, HDIM + 3 * NPTS, 1), lambda n, h: (h, 0, 0)),
            pl.BlockSpec((E, E), lambda n, h: (0, 0)),
            pl.BlockSpec((E, 1), lambda n, h: (0, 0)),
        ],
        out_specs=pl.BlockSpec((1, E, L), lambda n, h: (jnp.maximum(n - 1, 0), 0, 0)),
        out_shape=jax.ShapeDtypeStruct((N, E, L), jnp.float32),
        scratch_shapes=[pltpu.VMEM((2, E, L), jnp.bfloat16)],
        compiler_params=pltpu.CompilerParams(
            dimension_semantics=("arbitrary", "arbitrary")),
    )(qt, rpt, WcatT, bcat, WoutF, boutT)
    return outT.transpose(0, 2, 1).reshape(N, H, W, E)


# speedup vs baseline: 1.0734x; 1.0105x over previous
import jax
import jax.numpy as jnp
from jax.experimental import pallas as pl
from jax.experimental.pallas import tpu as pltpu

NHEADS = 12
NPTS = 8
HDIM = 64
CHUNK = 256


def _fused_kernel(qt_ref, rpt_ref, wcat_ref, bcat_ref, wout_ref, bout_ref,
                  out_ref, scr_ref):
    n = pl.program_id(0)
    h = pl.program_id(1)
    L = qt_ref.shape[2]
    gw = 32

    @pl.when(n < pl.num_programs(0) - 1)
    def _tents():
        qt = qt_ref[0]
        rt = jnp.dot(wcat_ref[0], qt, preferred_element_type=jnp.float32) + bcat_ref[0]
        valt = rt[0:HDIM, :]
        xft = rt[HDIM:HDIM + NPTS, :] + rpt_ref[0, 0:1, :]
        yft = rt[HDIM + NPTS:HDIM + 2 * NPTS, :] + rpt_ref[0, 1:2, :]
        logits = rt[HDIM + 2 * NPTS:HDIM + 3 * NPTS, :]
        m = jnp.max(logits, axis=0, keepdims=True)
        e = jnp.exp(logits - m)
        attnt = e / jnp.sum(e, axis=0, keepdims=True)

        g = jax.lax.broadcasted_iota(jnp.int32, (gw, 1), 0).astype(jnp.float32)
        txa_list = []
        ty_list = []
        for p in range(NPTS):
            ap = attnt[p:p + 1, :]
            txa = jnp.maximum(ap - ap * jnp.abs(g - xft[p:p + 1, :]), 0.0)
            ty = jnp.maximum(1.0 - jnp.abs(g - yft[p:p + 1, :]), 0.0)
            txa_list.append(txa.astype(jnp.bfloat16))
            ty_list.append(ty.astype(jnp.bfloat16))
        valb = valt.astype(jnp.bfloat16)
        slabs = []
        for j in range(0, L // gw, 2):
            s0 = None
            s1 = None
            for p in range(NPTS):
                x = txa_list[p]
                t0 = x * ty_list[p][j:j + 1, :]
                t1 = x * ty_list[p][j + 1:j + 2, :]
                s0 = t0 if s0 is None else s0 + t0
                s1 = t1 if s1 is None else s1 + t1
            slabs.append(s0)
            slabs.append(s1)
        acc = jnp.concatenate(slabs, axis=0)
        sampledt = jnp.dot(valb, acc, preferred_element_type=jnp.float32)
        scr_ref[jax.lax.rem(n, 2), pl.ds(h * HDIM, HDIM), :] = (
            sampledt.astype(jnp.bfloat16))

    @pl.when((n > 0) & (h == 0))
    def _project():
        prev = scr_ref[jax.lax.rem(n + 1, 2)]
        outv = jnp.dot(wout_ref[...], prev, preferred_element_type=jnp.float32)
        out_ref[0] = outv + bout_ref[...]


def kernel(query, reference_points, W_off, b_off, W_attn, b_attn, W_val, b_val, W_out, b_out):
    N, H, W, E = query.shape
    L = H * W
    qt = query.reshape(N, L, E).transpose(0, 2, 1)
    Wv = W_val.reshape(E, NHEADS, HDIM).transpose(1, 0, 2)
    Wo2 = W_off.reshape(E, NHEADS, NPTS, 2)
    Wox = float(W) * Wo2[..., 0].transpose(1, 0, 2)
    Woy = float(H) * Wo2[..., 1].transpose(1, 0, 2)
    Wa = W_attn.reshape(E, NHEADS, NPTS).transpose(1, 0, 2)
    Wcat = jnp.concatenate([Wv, Wox, Woy, Wa], axis=2)
    WcatT = Wcat.transpose(0, 2, 1)
    bo2 = b_off.reshape(NHEADS, NPTS, 2)
    bcat = jnp.concatenate([b_val.reshape(NHEADS, HDIM),
                            float(W) * bo2[..., 0], float(H) * bo2[..., 1],
                            b_attn.reshape(NHEADS, NPTS)], axis=1)[:, :, None]
    rpt = (reference_points.reshape(N, L, 2) * jnp.array([W, H], jnp.float32)
           - 0.5).transpose(0, 2, 1)
    WoutF = W_out.T.astype(jnp.bfloat16)
    boutT = b_out.reshape(E, 1)

    outT = pl.pallas_call(
        _fused_kernel,
        grid=(N + 1, NHEADS),
        in_specs=[
            pl.BlockSpec((1, E, L), lambda n, h: (jnp.minimum(n, N - 1), 0, 0)),
            pl.BlockSpec((1, 2, L), lambda n, h: (jnp.minimum(n, N - 1), 0, 0)),
            pl.BlockSpec((1, HDIM + 3 * NPTS, E), lambda n, h: (h, 0, 0)),
            pl.BlockSpec((1, HDIM + 3 * NPTS, 1), lambda n, h: (h, 0, 0)),
            pl.BlockSpec((E, E), lambda n, h: (0, 0)),
            pl.BlockSpec((E, 1), lambda n, h: (0, 0)),
        ],
        out_specs=pl.BlockSpec((1, E, L), lambda n, h: (jnp.maximum(n - 1, 0), 0, 0)),
        out_shape=jax.ShapeDtypeStruct((N, E, L), jnp.float32),
        scratch_shapes=[pltpu.VMEM((2, E, L), jnp.bfloat16)],
        compiler_params=pltpu.CompilerParams(
            dimension_semantics=("arbitrary", "arbitrary")),
    )(qt, rpt, WcatT, bcat, WoutF, boutT)
    return outT.transpose(0, 2, 1).reshape(N, H, W, E)
